# baseline (speedup 1.0000x reference)
import functools

import jax
import jax.numpy as jnp
from jax import lax
from jax.experimental import pallas as pl
from jax.experimental.pallas import tpu as pltpu
from jax.experimental.pallas import tpu_sc as plsc

NUM_E = 8
TOPK = 2
HID = 2048
HIDP = HID // 2
FFN_D = 4096

BS = 512
FT = 512
NJ = FFN_D // FT
BT = 512
BC = 512

SC_NC = 2
SC_NS = 16
NW = SC_NC * SC_NS
CH = 16
NBUF = 7


def _pack_bf16(x):
    a = lax.bitcast_convert_type(x[:, :HIDP], jnp.uint32)
    b = lax.bitcast_convert_type(x[:, HIDP:], jnp.uint32)
    a = a + (jnp.uint32(0x7FFF) + ((a >> 16) & jnp.uint32(1)))
    b = b + (jnp.uint32(0x7FFF) + ((b >> 16) & jnp.uint32(1)))
    p = (a & jnp.uint32(0xFFFF0000)) | (b >> 16)
    return lax.bitcast_convert_type(p, jnp.float32)


def _unpack_f32(p):
    u = lax.bitcast_convert_type(p, jnp.uint32)
    a = lax.bitcast_convert_type(u & jnp.uint32(0xFFFF0000), jnp.float32)
    b = lax.bitcast_convert_type(u << 16, jnp.float32)
    return a, b


def _router(hs, wg_pad, T):
    def body(x_ref, wg_ref, idx_ref, w_ref, xp_ref):
        x = x_ref[...]
        wg = wg_ref[...]
        logits = lax.dot_general(
            x, wg, (((1,), (1,)), ((), ())),
            preferred_element_type=jnp.float32)
        lane = lax.broadcasted_iota(jnp.int32, logits.shape, 1)
        neg = jnp.float32(-1e30)
        l = jnp.where(lane < NUM_E, logits, neg)
        m1 = jnp.max(l, axis=1, keepdims=True)
        e1 = jnp.min(jnp.where(l == m1, lane, 127), axis=1, keepdims=True)
        l2 = jnp.where(lane == e1, neg, l)
        m2 = jnp.max(l2, axis=1, keepdims=True)
        e2 = jnp.min(jnp.where(l2 == m2, lane, 127), axis=1, keepdims=True)
        w0 = 1.0 / (1.0 + jnp.exp(m2 - m1))
        w1v = 1.0 - w0
        idx_ref[...] = jnp.where(lane == 0, e1, jnp.where(lane == 1, e2, 0))
        w_ref[...] = jnp.where(lane == 0, w0, jnp.where(lane == 1, w1v, 0.0))
        xp_ref[...] = _pack_bf16(x)

    return pl.pallas_call(
        body,
        grid=(T // BT,),
        in_specs=[
            pl.BlockSpec((BT, HID), lambda b: (b, 0)),
            pl.BlockSpec((128, HID), lambda b: (0, 0)),
        ],
        out_specs=[
            pl.BlockSpec((BT, 128), lambda b: (b, 0)),
            pl.BlockSpec((BT, 128), lambda b: (b, 0)),
            pl.BlockSpec((BT, HIDP), lambda b: (b, 0)),
        ],
        out_shape=[
            jax.ShapeDtypeStruct((T, 128), jnp.int32),
            jax.ShapeDtypeStruct((T, 128), jnp.float32),
            jax.ShapeDtypeStruct((T, HIDP), jnp.float32),
        ],
    )(hs, wg_pad)


def _sc_gather(hs3, rows_sc, P):
    rpt = P // NW
    nch = rpt // CH
    mesh = plsc.VectorSubcoreMesh(core_axis_name="c", subcore_axis_name="s")

    @functools.partial(
        pl.kernel,
        mesh=mesh,
        out_type=jax.ShapeDtypeStruct((P, 8, 128), jnp.float32),
        scratch_types=(
            [pltpu.VMEM((nch, CH), jnp.int32)]
            + [pltpu.VMEM((CH, 8, 128), jnp.float32)] * NBUF
            + [pltpu.SemaphoreType.DMA] * (2 * NBUF)
        ),
    )
    def k(hs_hbm, rows_hbm, out_hbm, idx_v, *rest):
        bufs = rest[:NBUF]
        gsem = rest[NBUF:2 * NBUF]
        wsem = rest[2 * NBUF:3 * NBUF]
        wid = lax.axis_index("s") * SC_NC + lax.axis_index("c")
        base = wid * rpt
        pltpu.sync_copy(rows_hbm.at[wid], idx_v)

        gh = [None] * NBUF
        wh = [None] * NBUF
        for c in range(nch + NBUF - 1):
            if c < nch:
                s = c % NBUF
                if wh[s] is not None:
                    wh[s].wait()
                    wh[s] = None
                gh[s] = pltpu.async_copy(
                    hs_hbm.at[idx_v.at[c]], bufs[s], gsem[s])
            d = c - (NBUF - 1)
            if d >= 0:
                sd = d % NBUF
                gh[sd].wait()
                wh[sd] = pltpu.async_copy(
                    bufs[sd], out_hbm.at[pl.ds(base + d * CH, CH)],
                    wsem[sd])
        for s in range(NBUF):
            if wh[s] is not None:
                wh[s].wait()

    return k(hs3, rows_sc)


def _sc_scatter(ys3, dest_sc, P, T):
    rpt = P // NW
    nch = rpt // CH
    mesh = plsc.VectorSubcoreMesh(core_axis_name="c", subcore_axis_name="s")

    @functools.partial(
        pl.kernel,
        mesh=mesh,
        out_type=jax.ShapeDtypeStruct((2 * T + 8, 8, 128), jnp.float32),
        scratch_types=(
            [pltpu.VMEM((nch, CH), jnp.int32)]
            + [pltpu.VMEM((CH, 8, 128), jnp.float32)] * NBUF
            + [pltpu.SemaphoreType.DMA] * (2 * NBUF)
        ),
    )
    def k(ys_hbm, dest_hbm, buf_hbm, idx_v, *rest):
        bufs = rest[:NBUF]
        lsem = rest[NBUF:2 * NBUF]
        ssem = rest[2 * NBUF:3 * NBUF]
        wid = lax.axis_index("s") * SC_NC + lax.axis_index("c")
        base = wid * rpt
        pltpu.sync_copy(dest_hbm.at[wid], idx_v)

        lh = [None] * NBUF
        sh = [None] * NBUF
        for c in range(nch + NBUF - 1):
            if c < nch:
                s = c % NBUF
                if sh[s] is not None:
                    sh[s].wait()
                    sh[s] = None
                lh[s] = pltpu.async_copy(
                    ys_hbm.at[pl.ds(base + c * CH, CH)], bufs[s], lsem[s])
            d = c - (NBUF - 1)
            if d >= 0:
                sd = d % NBUF
                lh[sd].wait()
                sh[sd] = pltpu.async_copy(
                    bufs[sd], buf_hbm.at[idx_v.at[d]], ssem[sd])
        for s in range(NBUF):
            if sh[s] is not None:
                sh[s].wait()

    return k(ys3, dest_sc)


def _gmm(xp_sorted, W1, W2, W3, be, valid, P):
    nb = P // BS

    def body(be_ref, valid_ref, x_ref, w1_ref, w3_ref, w2_ref, out_ref,
             acc_ref):
        j = pl.program_id(1)

        @pl.when(valid_ref[pl.program_id(0)] == 1)
        def _():
            xa, xb = _unpack_f32(x_ref[...])
            x = jnp.concatenate(
                [xa.astype(jnp.bfloat16), xb.astype(jnp.bfloat16)], axis=1)
            w1 = w1_ref[0].astype(jnp.bfloat16)
            w3 = w3_ref[0].astype(jnp.bfloat16)
            a = lax.dot_general(x, w1, (((1,), (1,)), ((), ())),
                                preferred_element_type=jnp.float32)
            g = lax.dot_general(x, w3, (((1,), (1,)), ((), ())),
                                preferred_element_type=jnp.float32)
            h = (jnp.maximum(a, 0.0) * g).astype(jnp.bfloat16)
            w2 = w2_ref[0].astype(jnp.bfloat16)
            o = lax.dot_general(h, w2, (((1,), (1,)), ((), ())),
                                preferred_element_type=jnp.float32)

            @pl.when(j == 0)
            def _():
                acc_ref[...] = o

            @pl.when(j > 0)
            def _():
                acc_ref[...] += o

            @pl.when(j == NJ - 1)
            def _():
                out_ref[...] = _pack_bf16(acc_ref[...])

    grid_spec = pltpu.PrefetchScalarGridSpec(
        num_scalar_prefetch=2,
        grid=(nb, NJ),
        in_specs=[
            pl.BlockSpec((BS, HIDP), lambda b, j, be, valid: (b, 0)),
            pl.BlockSpec((1, FT, HID), lambda b, j, be, valid: (be[b], j, 0)),
            pl.BlockSpec((1, FT, HID), lambda b, j, be, valid: (be[b], j, 0)),
            pl.BlockSpec((1, HID, FT), lambda b, j, be, valid: (be[b], 0, j)),
        ],
        out_specs=pl.BlockSpec((BS, HIDP), lambda b, j, be, valid: (b, 0)),
        scratch_shapes=[pltpu.VMEM((BS, HID), jnp.float32)],
    )
    return pl.pallas_call(
        body,
        grid_spec=grid_spec,
        out_shape=jax.ShapeDtypeStruct((P, HIDP), jnp.float32),
        compiler_params=pltpu.CompilerParams(
            dimension_semantics=("arbitrary", "arbitrary")),
    )(be, valid, xp_sorted, W1, W3, W2)


def _combine(buf, w0col, w1col, T):
    def body(y0_ref, y1_ref, w0_ref, w1_ref, out_ref):
        a0, b0 = _unpack_f32(y0_ref[...])
        a1, b1 = _unpack_f32(y1_ref[...])
        w0 = w0_ref[...]
        w1 = w1_ref[...]
        out_ref[...] = jnp.concatenate(
            [a0 * w0 + a1 * w1, b0 * w0 + b1 * w1], axis=1)

    return pl.pallas_call(
        body,
        grid=(T // BC,),
        in_specs=[
            pl.BlockSpec((BC, HIDP), lambda b: (b, 0)),
            pl.BlockSpec((BC, HIDP), lambda b: (b + T // BC, 0)),
            pl.BlockSpec((BC, 1), lambda b: (b, 0)),
            pl.BlockSpec((BC, 1), lambda b: (b, 0)),
        ],
        out_specs=pl.BlockSpec((BC, HID), lambda b: (b, 0)),
        out_shape=jax.ShapeDtypeStruct((T, HID), jnp.float32),
    )(buf, buf, w0col, w1col)


def _index_prep(e_top, T, P):
    A = T * TOPK
    nb = P // BS
    ef = e_top.reshape(-1).astype(jnp.int32)
    ar = jnp.arange(A, dtype=jnp.int32)
    tok = ar // TOPK
    slot = ar % TOPK
    order = jnp.argsort(ef)
    es = ef[order]
    ts = tok[order]
    ss = slot[order]
    counts = jnp.bincount(ef, length=NUM_E).astype(jnp.int32)
    pc = ((counts + BS - 1) // BS) * BS
    cpc = jnp.cumsum(pc)
    pstart = cpc - pc
    cstart = jnp.cumsum(counts) - counts
    pos = pstart[es] + ar - cstart[es]
    rows = jnp.zeros((P,), jnp.int32).at[pos].set(ts)
    dest = jnp.full((P,), 2 * T, jnp.int32).at[pos].set(ss * T + ts)
    np_total = cpc[-1]
    bidx = jnp.arange(nb, dtype=jnp.int32)
    be = jnp.searchsorted(cpc, bidx * BS, side="right").astype(jnp.int32)
    e_last = jnp.searchsorted(cpc, np_total - 1, side="right").astype(jnp.int32)
    valid = (bidx * BS < np_total).astype(jnp.int32)
    be = jnp.where(valid == 1, jnp.minimum(be, NUM_E - 1), e_last)
    return rows, dest, be, valid


def kernel(hidden_states, W_gate, W1, W2, W3):
    batch, seq, dim = hidden_states.shape
    T = batch * seq
    P = T * TOPK + NUM_E * BS
    hs = hidden_states.reshape(T, dim)

    wg_pad = jnp.zeros((128, dim), jnp.float32).at[:NUM_E].set(W_gate)
    idx_out, w_out, xp = _router(hs, wg_pad, T)

    rows, dest, be, valid = _index_prep(idx_out[:, :TOPK], T, P)
    rows_sc = rows.reshape(NW, -1, CH)
    dest_sc = dest.reshape(NW, -1, CH)

    xp3 = xp.reshape(T, 8, 128)
    xp_sorted = _sc_gather(xp3, rows_sc, P).reshape(P, HIDP)
    ys = _gmm(xp_sorted, W1, W2, W3, be, valid, P)
    buf = _sc_scatter(ys.reshape(P, 8, 128), dest_sc, P, T)
    out = _combine(buf.reshape(2 * T + 8, HIDP), w_out[:, 0:1],
                   w_out[:, 1:2], T)
    return out.reshape(batch, seq, dim)


# speedup vs baseline: 1.1950x; 1.1950x over previous
import functools

import jax
import jax.numpy as jnp
from jax import lax
from jax.experimental import pallas as pl
from jax.experimental.pallas import tpu as pltpu
from jax.experimental.pallas import tpu_sc as plsc

NUM_E = 8
TOPK = 2
HID = 2048
HIDP = HID // 2
FFN_D = 4096

BS = 512
FT = 512
NJ = FFN_D // FT
BT = 512
BC = 512

SC_NC = 2
SC_NS = 16
NW = SC_NC * SC_NS
CH = 16
NBUF = 7


def _pack_bf16(x):
    a = lax.bitcast_convert_type(x[:, :HIDP], jnp.uint32)
    b = lax.bitcast_convert_type(x[:, HIDP:], jnp.uint32)
    a = a + (jnp.uint32(0x7FFF) + ((a >> 16) & jnp.uint32(1)))
    b = b + (jnp.uint32(0x7FFF) + ((b >> 16) & jnp.uint32(1)))
    p = (a & jnp.uint32(0xFFFF0000)) | (b >> 16)
    return lax.bitcast_convert_type(p, jnp.float32)


def _unpack_f32(p):
    u = lax.bitcast_convert_type(p, jnp.uint32)
    a = lax.bitcast_convert_type(u & jnp.uint32(0xFFFF0000), jnp.float32)
    b = lax.bitcast_convert_type(u << 16, jnp.float32)
    return a, b


def _router(hs, wg_pad, T):
    def body(x_ref, wg_ref, idx_ref, w_ref, xp_ref):
        x = x_ref[...]
        wg = wg_ref[...]
        logits = lax.dot_general(
            x, wg, (((1,), (1,)), ((), ())),
            preferred_element_type=jnp.float32)
        lane = lax.broadcasted_iota(jnp.int32, logits.shape, 1)
        neg = jnp.float32(-1e30)
        l = jnp.where(lane < NUM_E, logits, neg)
        m1 = jnp.max(l, axis=1, keepdims=True)
        e1 = jnp.min(jnp.where(l == m1, lane, 127), axis=1, keepdims=True)
        l2 = jnp.where(lane == e1, neg, l)
        m2 = jnp.max(l2, axis=1, keepdims=True)
        e2 = jnp.min(jnp.where(l2 == m2, lane, 127), axis=1, keepdims=True)
        w0 = 1.0 / (1.0 + jnp.exp(m2 - m1))
        w1v = 1.0 - w0
        idx_ref[...] = jnp.where(lane == 0, e1, jnp.where(lane == 1, e2, 0))
        w_ref[...] = jnp.where(lane == 0, w0, jnp.where(lane == 1, w1v, 0.0))
        xp_ref[...] = _pack_bf16(x)

    return pl.pallas_call(
        body,
        grid=(T // BT,),
        in_specs=[
            pl.BlockSpec((BT, HID), lambda b: (b, 0)),
            pl.BlockSpec((128, HID), lambda b: (0, 0)),
        ],
        out_specs=[
            pl.BlockSpec((BT, 128), lambda b: (b, 0)),
            pl.BlockSpec((BT, 128), lambda b: (b, 0)),
            pl.BlockSpec((BT, HIDP), lambda b: (b, 0)),
        ],
        out_shape=[
            jax.ShapeDtypeStruct((T, 128), jnp.int32),
            jax.ShapeDtypeStruct((T, 128), jnp.float32),
            jax.ShapeDtypeStruct((T, HIDP), jnp.float32),
        ],
    )(hs, wg_pad)


def _sc_gather(hs3, rows_sc, P):
    rpt = P // NW
    nch = rpt // CH
    mesh = plsc.VectorSubcoreMesh(core_axis_name="c", subcore_axis_name="s")

    @functools.partial(
        pl.kernel,
        mesh=mesh,
        out_type=jax.ShapeDtypeStruct((P, 8, 128), jnp.float32),
        scratch_types=(
            [pltpu.VMEM((nch, CH), jnp.int32)]
            + [pltpu.VMEM((CH, 8, 128), jnp.float32)] * NBUF
            + [pltpu.SemaphoreType.DMA] * (2 * NBUF)
        ),
    )
    def k(hs_hbm, rows_hbm, out_hbm, idx_v, *rest):
        bufs = rest[:NBUF]
        gsem = rest[NBUF:2 * NBUF]
        wsem = rest[2 * NBUF:3 * NBUF]
        wid = lax.axis_index("s") * SC_NC + lax.axis_index("c")
        base = wid * rpt
        pltpu.sync_copy(rows_hbm.at[wid], idx_v)

        gh = [None] * NBUF
        wh = [None] * NBUF
        for c in range(nch + NBUF - 1):
            if c < nch:
                s = c % NBUF
                if wh[s] is not None:
                    wh[s].wait()
                    wh[s] = None
                gh[s] = pltpu.async_copy(
                    hs_hbm.at[idx_v.at[c]], bufs[s], gsem[s])
            d = c - (NBUF - 1)
            if d >= 0:
                sd = d % NBUF
                gh[sd].wait()
                wh[sd] = pltpu.async_copy(
                    bufs[sd], out_hbm.at[pl.ds(base + d * CH, CH)],
                    wsem[sd])
        for s in range(NBUF):
            if wh[s] is not None:
                wh[s].wait()

    return k(hs3, rows_sc)


def _gmm(xp_sorted, W1, W2, W3, be, valid, P):
    nb = P // BS

    def body(be_ref, valid_ref, x_ref, w1_ref, w3_ref, w2_ref, out_ref,
             acc_ref):
        j = pl.program_id(1)

        @pl.when(valid_ref[pl.program_id(0)] == 1)
        def _():
            xa, xb = _unpack_f32(x_ref[...])
            x = jnp.concatenate(
                [xa.astype(jnp.bfloat16), xb.astype(jnp.bfloat16)], axis=1)
            w1 = w1_ref[0].astype(jnp.bfloat16)
            w3 = w3_ref[0].astype(jnp.bfloat16)
            a = lax.dot_general(x, w1, (((1,), (1,)), ((), ())),
                                preferred_element_type=jnp.float32)
            g = lax.dot_general(x, w3, (((1,), (1,)), ((), ())),
                                preferred_element_type=jnp.float32)
            h = (jnp.maximum(a, 0.0) * g).astype(jnp.bfloat16)
            w2 = w2_ref[0].astype(jnp.bfloat16)
            o = lax.dot_general(h, w2, (((1,), (1,)), ((), ())),
                                preferred_element_type=jnp.float32)

            @pl.when(j == 0)
            def _():
                acc_ref[...] = o

            @pl.when(j > 0)
            def _():
                acc_ref[...] += o

            @pl.when(j == NJ - 1)
            def _():
                out_ref[...] = _pack_bf16(acc_ref[...])

    grid_spec = pltpu.PrefetchScalarGridSpec(
        num_scalar_prefetch=2,
        grid=(nb, NJ),
        in_specs=[
            pl.BlockSpec((BS, HIDP), lambda b, j, be, valid: (b, 0)),
            pl.BlockSpec((1, FT, HID), lambda b, j, be, valid: (be[b], j, 0)),
            pl.BlockSpec((1, FT, HID), lambda b, j, be, valid: (be[b], j, 0)),
            pl.BlockSpec((1, HID, FT), lambda b, j, be, valid: (be[b], 0, j)),
        ],
        out_specs=pl.BlockSpec((BS, HIDP), lambda b, j, be, valid: (b, 0)),
        scratch_shapes=[pltpu.VMEM((BS, HID), jnp.float32)],
    )
    return pl.pallas_call(
        body,
        grid_spec=grid_spec,
        out_shape=jax.ShapeDtypeStruct((P, HIDP), jnp.float32),
        compiler_params=pltpu.CompilerParams(
            dimension_semantics=("arbitrary", "arbitrary")),
    )(be, valid, xp_sorted, W1, W3, W2)


def _combine(buf, w0col, w1col, T):
    def body(y0_ref, y1_ref, w0_ref, w1_ref, out_ref):
        a0, b0 = _unpack_f32(y0_ref[...])
        a1, b1 = _unpack_f32(y1_ref[...])
        w0 = w0_ref[...]
        w1 = w1_ref[...]
        out_ref[...] = jnp.concatenate(
            [a0 * w0 + a1 * w1, b0 * w0 + b1 * w1], axis=1)

    return pl.pallas_call(
        body,
        grid=(T // BC,),
        in_specs=[
            pl.BlockSpec((BC, HIDP), lambda b: (b, 0)),
            pl.BlockSpec((BC, HIDP), lambda b: (b + T // BC, 0)),
            pl.BlockSpec((BC, 1), lambda b: (b, 0)),
            pl.BlockSpec((BC, 1), lambda b: (b, 0)),
        ],
        out_specs=pl.BlockSpec((BC, HID), lambda b: (b, 0)),
        out_shape=jax.ShapeDtypeStruct((T, HID), jnp.float32),
    )(buf, buf, w0col, w1col)


def _index_prep(e_top, T, P):
    A = T * TOPK
    nb = P // BS
    ef = e_top.reshape(-1).astype(jnp.int32)
    ar = jnp.arange(A, dtype=jnp.int32)
    tok = ar // TOPK
    order = jnp.argsort(ef)
    es = ef[order]
    ts = tok[order]
    counts = jnp.bincount(ef, length=NUM_E).astype(jnp.int32)
    pc = ((counts + BS - 1) // BS) * BS
    cpc = jnp.cumsum(pc)
    pstart = cpc - pc
    cstart = jnp.cumsum(counts) - counts
    pos = pstart[es] + ar - cstart[es]
    rows = jnp.zeros((P,), jnp.int32).at[pos].set(ts)
    pos_orig = jnp.zeros((A,), jnp.int32).at[order].set(pos)
    p01 = jnp.concatenate([pos_orig[0::2], pos_orig[1::2]])
    np_total = cpc[-1]
    bidx = jnp.arange(nb, dtype=jnp.int32)
    be = jnp.searchsorted(cpc, bidx * BS, side="right").astype(jnp.int32)
    e_last = jnp.searchsorted(cpc, np_total - 1, side="right").astype(jnp.int32)
    valid = (bidx * BS < np_total).astype(jnp.int32)
    be = jnp.where(valid == 1, jnp.minimum(be, NUM_E - 1), e_last)
    return rows, p01, be, valid


def kernel(hidden_states, W_gate, W1, W2, W3):
    batch, seq, dim = hidden_states.shape
    T = batch * seq
    P = T * TOPK + NUM_E * BS
    hs = hidden_states.reshape(T, dim)

    wg_pad = jnp.zeros((128, dim), jnp.float32).at[:NUM_E].set(W_gate)
    idx_out, w_out, xp = _router(hs, wg_pad, T)

    rows, p01, be, valid = _index_prep(idx_out[:, :TOPK], T, P)
    rows_sc = rows.reshape(NW, -1, CH)
    p01_sc = p01.reshape(NW, -1, CH)

    xp3 = xp.reshape(T, 8, 128)
    xp_sorted = _sc_gather(xp3, rows_sc, P).reshape(P, HIDP)
    ys = _gmm(xp_sorted, W1, W2, W3, be, valid, P)
    y01 = _sc_gather(ys.reshape(P, 8, 128), p01_sc, 2 * T)
    out = _combine(y01.reshape(2 * T, HIDP), w_out[:, 0:1],
                   w_out[:, 1:2], T)
    return out.reshape(batch, seq, dim)
